# baseline (speedup 1.0000x reference)
import functools

import jax
import jax.numpy as jnp
from jax import lax
from jax.experimental import pallas as pl
from jax.experimental.pallas import tpu as pltpu
from jax.experimental.pallas import tpu_sc as plsc

B = 64
N_TOK = 196
D = 768
POOL = 512
TOPK = 4
L = 5
DUAL = 2
ROW = 20 * 12 * 64
N_ROWS_OUT = L * B * DUAL * TOPK
N_ROWS_TABLE = L * DUAL * POOL

B_BLK = 8


def _sim_topk_body(x_ref, pk_ref, sim_ref, idx_ref):
    x_mean = jnp.mean(x_ref[...], axis=0)
    x_norm = x_mean * lax.rsqrt(
        jnp.maximum(jnp.sum(x_mean * x_mean, axis=-1, keepdims=True), 1e-12))
    pk = pk_ref[...]
    pk_norm = pk * lax.rsqrt(
        jnp.maximum(jnp.sum(pk * pk, axis=-1, keepdims=True), 1e-12))
    sim = jnp.dot(x_norm, pk_norm.T,
                  preferred_element_type=jnp.float32)
    sim_ref[...] = sim

    iota = lax.broadcasted_iota(jnp.int32, (B_BLK, POOL), 1)
    cur = sim
    cols = []
    for _ in range(TOPK):
        m = jnp.max(cur, axis=1, keepdims=True)
        j = jnp.min(jnp.where(cur == m, iota, POOL), axis=1)
        cols.append(j[:, None])
        cur = jnp.where(iota == j[:, None], jnp.float32(-jnp.inf), cur)
    idx_ref[...] = jnp.concatenate(cols, axis=1)


def _sim_topk(xv, prompt_key):
    return pl.pallas_call(
        _sim_topk_body,
        grid=(B // B_BLK,),
        in_specs=[
            pl.BlockSpec((N_TOK, B_BLK, D), lambda i: (0, i, 0)),
            pl.BlockSpec((POOL, D), lambda i: (0, 0)),
        ],
        out_specs=[
            pl.BlockSpec((B_BLK, POOL), lambda i: (i, 0)),
            pl.BlockSpec((B_BLK, TOPK), lambda i: (i, 0)),
        ],
        out_shape=[
            jax.ShapeDtypeStruct((B, POOL), jnp.float32),
            jax.ShapeDtypeStruct((B, TOPK), jnp.int32),
        ],
    )(xv, prompt_key)


T_BLK = 4
_TCOLS = T_BLK * 768
_NTC = 20 // T_BLK


def _transpose_body(x_ref, o_ref):
    x = x_ref[0, 0].reshape(_TCOLS, POOL)
    o_ref[0, 0, 0] = x.T


def _pool_major_table(ptv):
    return pl.pallas_call(
        _transpose_body,
        grid=(L, DUAL, _NTC),
        in_specs=[
            pl.BlockSpec((1, 1, T_BLK, 768, POOL),
                         lambda l, d, j: (l, d, j, 0, 0)),
        ],
        out_specs=pl.BlockSpec((1, 1, 1, POOL, _TCOLS),
                               lambda l, d, j: (l, d, j, 0, 0)),
        out_shape=jax.ShapeDtypeStruct((L, DUAL, _NTC, POOL, _TCOLS),
                                       jnp.float32),
    )(ptv)


_NW = 32
_NSUB = N_ROWS_OUT * _NTC
_TBLSUB = N_ROWS_TABLE * _NTC
_RPW = _NSUB // _NW
_CHUNK = 8
_NCHUNK = _RPW // _CHUNK


@functools.cache
def _sc_gather_fn():
    @functools.partial(
        pl.kernel,
        out_type=jax.ShapeDtypeStruct((_NSUB, _TCOLS), jnp.float32),
        mesh=plsc.VectorSubcoreMesh(core_axis_name="c", subcore_axis_name="s"),
        scratch_types=[
            pltpu.VMEM((_NCHUNK, _CHUNK), jnp.int32),
            pltpu.VMEM((2, _CHUNK, _TCOLS), jnp.float32),
            pltpu.SemaphoreType.DMA,
            pltpu.SemaphoreType.DMA,
            pltpu.SemaphoreType.DMA,
        ],
    )
    def _sc_gather(table_hbm, src_hbm, out_hbm, idx_v, buf, gsem0, gsem1,
                   osem):
        wid = lax.axis_index("s") * 2 + lax.axis_index("c")
        pltpu.sync_copy(src_hbm.at[wid], idx_v)
        base = wid * _RPW

        def gather(g, slot, sem):
            pltpu.async_copy(table_hbm.at[idx_v.at[g]], buf.at[slot], sem)

        def gather_wait(g, slot, sem):
            pltpu.make_async_copy(table_hbm.at[idx_v.at[g]], buf.at[slot],
                                  sem).wait()

        def out_copy(g, slot):
            pltpu.async_copy(
                buf.at[slot], out_hbm.at[pl.ds(base + g * _CHUNK, _CHUNK)],
                osem)

        def out_wait(g, slot):
            pltpu.make_async_copy(
                buf.at[slot], out_hbm.at[pl.ds(base + g * _CHUNK, _CHUNK)],
                osem).wait()

        gather(0, 0, gsem0)

        def body(g, _):
            slot = lax.rem(g, 2)
            nslot = 1 - slot
            nsem = lax.cond(slot == 0, lambda: 1, lambda: 0)

            @pl.when(g + 1 < _NCHUNK)
            def _():
                @pl.when(g >= 1)
                def _():
                    out_wait(g - 1, nslot)

                @pl.when(nsem == 1)
                def _():
                    gather(g + 1, nslot, gsem1)

                @pl.when(nsem == 0)
                def _():
                    gather(g + 1, nslot, gsem0)

            @pl.when(lax.rem(g, 2) == 0)
            def _():
                gather_wait(g, slot, gsem0)

            @pl.when(lax.rem(g, 2) == 1)
            def _():
                gather_wait(g, slot, gsem1)

            out_copy(g, slot)
            return 0

        lax.fori_loop(0, _NCHUNK, body, 0)
        out_wait(_NCHUNK - 2, (_NCHUNK - 2) % 2)
        out_wait(_NCHUNK - 1, (_NCHUNK - 1) % 2)

    return _sc_gather


def kernel(x_embed, prompt, prompt_key):
    xv = jnp.transpose(x_embed, (1, 0, 2))
    similarity, idx = _sim_topk(xv, prompt_key)

    ptv = jnp.transpose(prompt, (0, 1, 3, 4, 5, 2)).reshape(
        L, DUAL, 20, 768, POOL)
    table = _pool_major_table(ptv).reshape(_TBLSUB, _TCOLS)

    l_ = jnp.arange(L)[:, None, None, None, None]
    d_ = jnp.arange(DUAL)[None, None, :, None, None]
    tc = jnp.arange(_NTC)[None, None, None, None, :]
    src = ((l_ * DUAL + d_) * _NTC + tc) * POOL + idx[None, :, None, :, None]
    src = src.reshape(_NW, _NCHUNK, _CHUNK).astype(jnp.int32)

    gathered = _sc_gather_fn()(table, src)
    batched_prompt = gathered.reshape(L, B, DUAL, TOPK * 20, 12, 64)
    return (batched_prompt, similarity, idx)


# speedup vs baseline: 1.0668x; 1.0668x over previous
import functools

import jax
import jax.numpy as jnp
from jax import lax
from jax.experimental import pallas as pl
from jax.experimental.pallas import tpu as pltpu
from jax.experimental.pallas import tpu_sc as plsc

B = 64
N_TOK = 196
D = 768
POOL = 512
TOPK = 4
L = 5
DUAL = 2
ROW = 20 * 12 * 64
N_ROWS_OUT = L * B * DUAL * TOPK
N_ROWS_TABLE = L * DUAL * POOL

B_BLK = 8
P_BLK = 128


def _sim_topk_body(x_ref, pk_ref, sim_ref, idx_ref):
    x_mean = jnp.mean(x_ref[...], axis=0)
    x_norm = x_mean * lax.rsqrt(
        jnp.maximum(jnp.sum(x_mean * x_mean, axis=-1, keepdims=True), 1e-12))
    pk = pk_ref[...]
    pk_norm = pk * lax.rsqrt(
        jnp.maximum(jnp.sum(pk * pk, axis=-1, keepdims=True), 1e-12))
    sim = jnp.dot(x_norm, pk_norm.T,
                  preferred_element_type=jnp.float32)
    sim_ref[...] = sim

    iota = lax.broadcasted_iota(jnp.int32, (B_BLK, POOL), 1)
    cur = sim
    cols = []
    for _ in range(TOPK):
        m = jnp.max(cur, axis=1, keepdims=True)
        j = jnp.min(jnp.where(cur == m, iota, POOL), axis=1)
        cols.append(j[:, None])
        cur = jnp.where(iota == j[:, None], jnp.float32(-jnp.inf), cur)
    idx_ref[...] = jnp.concatenate(cols, axis=1)


def _sim_topk(xv, prompt_key):
    return pl.pallas_call(
        _sim_topk_body,
        grid=(B // B_BLK,),
        in_specs=[
            pl.BlockSpec((N_TOK, B_BLK, D), lambda i: (0, i, 0)),
            pl.BlockSpec((POOL, D), lambda i: (0, 0)),
        ],
        out_specs=[
            pl.BlockSpec((B_BLK, POOL), lambda i: (i, 0)),
            pl.BlockSpec((B_BLK, TOPK), lambda i: (i, 0)),
        ],
        out_shape=[
            jax.ShapeDtypeStruct((B, POOL), jnp.float32),
            jax.ShapeDtypeStruct((B, TOPK), jnp.int32),
        ],
    )(xv, prompt_key)


def _transpose_body(x_ref, o_ref):
    x = x_ref[0, 0].reshape(ROW, P_BLK)
    o_ref[0, 0] = x.T


def _pool_major_table(ptv):
    return pl.pallas_call(
        _transpose_body,
        grid=(L, DUAL, POOL // P_BLK),
        in_specs=[
            pl.BlockSpec((1, 1, 20, 768, P_BLK),
                         lambda l, d, j: (l, d, 0, 0, j)),
        ],
        out_specs=pl.BlockSpec((1, 1, P_BLK, ROW),
                               lambda l, d, j: (l, d, j, 0)),
        out_shape=jax.ShapeDtypeStruct((L, DUAL, POOL, ROW), jnp.float32),
    )(ptv)


_NW = 32
_RPW = N_ROWS_OUT // _NW
_CHUNK = 8
_NCHUNK = _RPW // _CHUNK


@functools.cache
def _sc_gather_fn():
    @functools.partial(
        pl.kernel,
        out_type=jax.ShapeDtypeStruct((N_ROWS_OUT, ROW), jnp.float32),
        mesh=plsc.VectorSubcoreMesh(core_axis_name="c", subcore_axis_name="s"),
        scratch_types=[
            pltpu.VMEM((_NCHUNK, _CHUNK), jnp.int32),
            pltpu.VMEM((_CHUNK, ROW), jnp.float32),
            pltpu.SemaphoreType.DMA,
        ],
    )
    def _sc_gather(table_hbm, src_hbm, out_hbm, idx_v, buf, sem):
        wid = lax.axis_index("s") * 2 + lax.axis_index("c")
        pltpu.sync_copy(src_hbm.at[wid], idx_v)
        base = wid * _RPW
        for g in range(_NCHUNK):
            pltpu.async_copy(table_hbm.at[idx_v.at[g]], buf, sem).wait()
            pltpu.sync_copy(buf, out_hbm.at[pl.ds(base + g * _CHUNK, _CHUNK)])

    return _sc_gather


def kernel(x_embed, prompt, prompt_key):
    xv = jnp.transpose(x_embed, (1, 0, 2))
    similarity, idx = _sim_topk(xv, prompt_key)

    ptv = jnp.transpose(prompt, (0, 1, 3, 4, 5, 2)).reshape(
        L, DUAL, 20, 768, POOL)
    table = _pool_major_table(ptv).reshape(N_ROWS_TABLE, ROW)

    ld = (jnp.arange(L)[:, None, None, None] * DUAL
          + jnp.arange(DUAL)[None, None, :, None])
    src = ld * POOL + idx[None, :, None, :]
    src = src.reshape(_NW, _NCHUNK, _CHUNK).astype(jnp.int32)

    gathered = _sc_gather_fn()(table, src)
    batched_prompt = gathered.reshape(L, B, DUAL, TOPK * 20, 12, 64)
    return (batched_prompt, similarity, idx)
